# baseline (speedup 1.0000x reference)
import functools

import jax
import jax.numpy as jnp
from jax import lax
from jax.experimental import pallas as pl
from jax.experimental.pallas import tpu as pltpu
from jax.experimental.pallas import tpu_sc as plsc

D = 32
NC = 2
NS = 16
NW = NC * NS
C = 640
G = 128
NB = 4


@functools.partial(jax.jit, static_argnames=("n_rows",))
def _gather_rows(idx, table, n_rows):
    b_per_w = n_rows // NW
    n_chunks = b_per_w // C
    n_groups = n_chunks // NB
    mesh = plsc.VectorSubcoreMesh(core_axis_name="c", subcore_axis_name="s")

    @functools.partial(
        pl.kernel,
        mesh=mesh,
        out_type=jax.ShapeDtypeStruct((n_rows, D), jnp.float32),
        scratch_types=[
            pltpu.VMEM((NB, C), jnp.int32),
            pltpu.VMEM((NB, C, D), jnp.float32),
            [pltpu.SemaphoreType.DMA] * NB,
            [pltpu.SemaphoreType.DMA] * NB,
        ],
        compiler_params=pltpu.CompilerParams(use_tc_tiling_on_sc=False),
    )
    def emb(idx_hbm, table_hbm, out_hbm, idx_v, rows_v, gsems, ssems):
        wid = lax.axis_index("s") * NC + lax.axis_index("c")
        base = wid * b_per_w

        def fire(chunk, b):
            off = base + chunk * C
            pltpu.sync_copy(idx_hbm.at[pl.ds(off, C)], idx_v.at[b])
            for j in range(C // G):
                pltpu.async_copy(
                    table_hbm.at[idx_v.at[b, pl.ds(j * G, G)]],
                    rows_v.at[b, pl.ds(j * G, G)],
                    gsems[b],
                )

        for b in range(NB):
            fire(b, b)

        def body(g, carry):
            for b in range(NB):
                chunk = g * NB + b
                off = base + chunk * C
                for j in range(C // G):
                    pltpu.make_async_copy(
                        table_hbm.at[idx_v.at[b, pl.ds(j * G, G)]],
                        rows_v.at[b, pl.ds(j * G, G)],
                        gsems[b],
                    ).wait()
                pltpu.async_copy(rows_v.at[b], out_hbm.at[pl.ds(off, C)], ssems[b])

            @pl.when(g < n_groups - 1)
            def _():
                for b in range(NB):
                    pltpu.make_async_copy(
                        rows_v.at[b], out_hbm.at[pl.ds(base, C)], ssems[b]
                    ).wait()
                    fire((g + 1) * NB + b, b)
            return carry

        lax.fori_loop(0, n_groups, body, 0)

        for b in range(NB):
            pltpu.make_async_copy(
                rows_v.at[b], out_hbm.at[pl.ds(base, C)], ssems[b]
            ).wait()

    return emb(idx, table)


def kernel(x, table):
    b, s = x.shape
    n_rows = b * s
    idx = jnp.maximum(
        jnp.reshape(jnp.transpose(x), (n_rows,)), 0
    ).astype(jnp.int32)
    vocab = table.shape[0]
    t4 = lax.optimization_barrier(jnp.reshape(table, (vocab * D // 128, 128)))
    out = _gather_rows(idx, jnp.reshape(t4, (vocab, D)), n_rows)
    return jnp.transpose(jnp.reshape(out, (s, b, D)), (1, 0, 2))


# speedup vs baseline: 1.0006x; 1.0006x over previous
import functools

import jax
import jax.numpy as jnp
from jax import lax
from jax.experimental import pallas as pl
from jax.experimental.pallas import tpu as pltpu
from jax.experimental.pallas import tpu_sc as plsc

D = 32
NC = 2
NS = 16
NW = NC * NS
C = 640
G = 128
NB = 4


@functools.partial(jax.jit, static_argnames=("n_rows",))
def _gather_rows(idx, table, n_rows):
    b_per_w = n_rows // NW
    n_chunks = b_per_w // C
    n_groups = n_chunks // NB
    mesh = plsc.VectorSubcoreMesh(core_axis_name="c", subcore_axis_name="s")

    @functools.partial(
        pl.kernel,
        mesh=mesh,
        out_type=jax.ShapeDtypeStruct((n_rows, D), jnp.float32),
        scratch_types=[
            pltpu.VMEM((NB, C), jnp.int32),
            pltpu.VMEM((NB, C, D), jnp.float32),
            [pltpu.SemaphoreType.DMA] * NB,
            [pltpu.SemaphoreType.DMA] * NB,
        ],
        compiler_params=pltpu.CompilerParams(use_tc_tiling_on_sc=False),
    )
    def emb(idx_hbm, table_hbm, out_hbm, idx_v, rows_v, gsems, ssems):
        wid = lax.axis_index("s") * NC + lax.axis_index("c")
        base = wid * b_per_w

        def fire(chunk, b):
            off = base + chunk * C
            pltpu.sync_copy(idx_hbm.at[pl.ds(off, C)], idx_v.at[b])
            for j in range(C // G):
                pltpu.async_copy(
                    table_hbm.at[idx_v.at[b, pl.ds(j * G, G)]],
                    rows_v.at[b, pl.ds(j * G, G)],
                    gsems[b],
                )

        for b in range(NB):
            fire(b, b)

        def body(g, carry):
            for b in range(NB):
                chunk = g * NB + b
                off = base + chunk * C
                for j in range(C // G):
                    pltpu.make_async_copy(
                        table_hbm.at[idx_v.at[b, pl.ds(j * G, G)]],
                        rows_v.at[b, pl.ds(j * G, G)],
                        gsems[b],
                    ).wait()
                pltpu.async_copy(rows_v.at[b], out_hbm.at[pl.ds(off, C)], ssems[b])

            @pl.when(g < n_groups - 1)
            def _():
                for b in range(NB):
                    pltpu.make_async_copy(
                        rows_v.at[b], out_hbm.at[pl.ds(base, C)], ssems[b]
                    ).wait()
                    fire((g + 1) * NB + b, b)
            return carry

        lax.fori_loop(0, n_groups, body, 0)

        for b in range(NB):
            pltpu.make_async_copy(
                rows_v.at[b], out_hbm.at[pl.ds(base, C)], ssems[b]
            ).wait()

    return emb(idx, table)


def kernel(x, table):
    b, s = x.shape
    n_rows = b * s
    idx = jnp.maximum(
        jnp.reshape(jnp.transpose(x), (n_rows,)), 0
    ).astype(jnp.int32)
    out = _gather_rows(idx, table, n_rows)
    return jnp.transpose(jnp.reshape(out, (s, b, D)), (1, 0, 2))
